# baseline (speedup 1.0000x reference)
import functools

import jax
import jax.numpy as jnp
from jax import lax
from jax.experimental import pallas as pl
from jax.experimental.pallas import tpu as pltpu
from jax.experimental.pallas import tpu_sc as plsc

_N = 10000
_E = 320000
_F = 128
_NP = 10240
_NC, _NS, _LN = 2, 16, 16
_NW = _NC * _NS
_EPW = _E // _NW
_CH = 80
_NCH = _EPW // _CH
_GRP = _CH // _LN
_PPT = _NP // _NS
_RPT = _NP // _NS
_ZB = 16

_MESH = plsc.VectorSubcoreMesh(core_axis_name="c", subcore_axis_name="s")
_SC_PARAMS = pltpu.CompilerParams(needs_layout_passes=False)


def _zero_vec_ref(ref, n_vec):
  for i in range(n_vec):
    ref[pl.ds(i * _LN, _LN)] = jnp.zeros((_LN,), jnp.float32)


@functools.partial(
    pl.kernel,
    out_type=jax.ShapeDtypeStruct((2, _NP), jnp.float32),
    mesh=_MESH,
    compiler_params=_SC_PARAMS,
    scratch_types=[
        pltpu.VMEM((3, _CH), jnp.int32),
        pltpu.VMEM((3, _CH), jnp.float32),
        pltpu.VMEM((_PPT,), jnp.float32),
        pltpu.VMEM_SHARED((_NP,), jnp.float32),
        pltpu.SemaphoreType.DMA,
        pltpu.SemaphoreType.DMA,
        pltpu.SemaphoreType.DMA,
        pltpu.SemaphoreType.DMA,
    ],
)
def _deg(dst_hbm, ew_hbm, out_hbm, dbuf, ebuf, zb, deg_sh, isem,
         ssem0, ssem1, ssem2):
  c = lax.axis_index("c")
  s = lax.axis_index("s")
  wid = s * _NC + c
  base0 = wid * _EPW
  _zero_vec_ref(zb, _PPT // _LN)
  pltpu.sync_copy(zb, deg_sh.at[pl.ds(s * _PPT, _PPT)])
  plsc.subcore_barrier()

  def fetch(i, slot):
    pltpu.async_copy(dst_hbm.at[pl.ds(base0 + i * _CH, _CH)],
                     dbuf.at[slot], isem)
    pltpu.async_copy(ew_hbm.at[pl.ds(base0 + i * _CH, _CH)],
                     ebuf.at[slot], isem)

  def wait_fetch(i, slot):
    pltpu.make_async_copy(dst_hbm.at[pl.ds(base0 + i * _CH, _CH)],
                          dbuf.at[slot], isem).wait()
    pltpu.make_async_copy(ew_hbm.at[pl.ds(base0 + i * _CH, _CH)],
                          ebuf.at[slot], isem).wait()

  def start_scatter(slot):
    for k, sem in ((0, ssem0), (1, ssem1), (2, ssem2)):
      @pl.when(slot == k)
      def _(k=k, sem=sem):
        pltpu.async_copy(ebuf.at[k], deg_sh.at[dbuf.at[k]], sem, add=True)

  def wait_scatter(slot):
    for k, sem in ((0, ssem0), (1, ssem1), (2, ssem2)):
      @pl.when(slot == k)
      def _(k=k, sem=sem):
        pltpu.make_async_copy(ebuf.at[k], deg_sh.at[dbuf.at[k]], sem).wait()

  fetch(0, 0)
  fetch(1, 1)

  def body(i, carry):
    slot = lax.rem(i, 3)

    @pl.when(i >= 1)
    def _():
      wait_scatter(lax.rem(i + 2, 3))

    wait_fetch(i, slot)

    @pl.when(i + 2 < _NCH)
    def _():
      fetch(i + 2, lax.rem(i + 2, 3))

    start_scatter(slot)
    return carry

  lax.fori_loop(0, _NCH, body, 0)
  wait_scatter(jnp.int32((_NCH - 1) % 3))
  plsc.subcore_barrier()
  pltpu.sync_copy(deg_sh.at[pl.ds(s * _PPT, _PPT)],
                  out_hbm.at[c, pl.ds(s * _PPT, _PPT)])


@functools.partial(
    pl.kernel,
    out_type=jax.ShapeDtypeStruct((2, _NP, _F), jnp.float32),
    mesh=_MESH,
    compiler_params=_SC_PARAMS,
    scratch_types=[
        pltpu.VMEM((3, _CH), jnp.int32),
        pltpu.VMEM((3, _CH), jnp.int32),
        pltpu.VMEM((3, _CH), jnp.float32),
        pltpu.VMEM((2, _CH, _F), jnp.float32),
        pltpu.VMEM((_NP,), jnp.float32),
        pltpu.VMEM((_ZB, _F), jnp.float32),
        pltpu.VMEM_SHARED((_NP, _F), jnp.float32),
        pltpu.SemaphoreType.DMA,
        pltpu.SemaphoreType.DMA,
        pltpu.SemaphoreType.DMA,
        pltpu.SemaphoreType.DMA,
        pltpu.SemaphoreType.DMA,
    ],
)
def _agg(tab_hbm, src_hbm, dst_hbm, ew_hbm, dinv_hbm, out_hbm,
         sbuf, dbuf, ebuf, rows, dinv, zrows, acc_sh, isem,
         gsem0, gsem1, ssem0, ssem1):
  c = lax.axis_index("c")
  s = lax.axis_index("s")
  wid = s * _NC + c
  base0 = wid * _EPW

  for r in range(_ZB):
    for k in range(_F // _LN):
      zrows[r, pl.ds(k * _LN, _LN)] = jnp.zeros((_LN,), jnp.float32)

  def zbody(i, carry):
    pltpu.sync_copy(zrows, acc_sh.at[pl.ds(s * _RPT + i * _ZB, _ZB)])
    return carry

  lax.fori_loop(0, _RPT // _ZB, zbody, 0)

  pltpu.sync_copy(dinv_hbm, dinv)
  plsc.subcore_barrier()

  def fetch_idx(i, slot):
    pltpu.async_copy(src_hbm.at[pl.ds(base0 + i * _CH, _CH)],
                     sbuf.at[slot], isem)
    pltpu.async_copy(dst_hbm.at[pl.ds(base0 + i * _CH, _CH)],
                     dbuf.at[slot], isem)
    pltpu.async_copy(ew_hbm.at[pl.ds(base0 + i * _CH, _CH)],
                     ebuf.at[slot], isem)

  def wait_idx(i, slot):
    pltpu.make_async_copy(src_hbm.at[pl.ds(base0 + i * _CH, _CH)],
                          sbuf.at[slot], isem).wait()
    pltpu.make_async_copy(dst_hbm.at[pl.ds(base0 + i * _CH, _CH)],
                          dbuf.at[slot], isem).wait()
    pltpu.make_async_copy(ew_hbm.at[pl.ds(base0 + i * _CH, _CH)],
                          ebuf.at[slot], isem).wait()

  def start_gather(islot, rslot):
    @pl.when(rslot == 0)
    def _():
      pltpu.async_copy(tab_hbm.at[sbuf.at[islot]], rows.at[0], gsem0)

    @pl.when(rslot == 1)
    def _():
      pltpu.async_copy(tab_hbm.at[sbuf.at[islot]], rows.at[1], gsem1)

  def wait_gather(islot, rslot):
    @pl.when(rslot == 0)
    def _():
      pltpu.make_async_copy(tab_hbm.at[sbuf.at[islot]], rows.at[0],
                            gsem0).wait()

    @pl.when(rslot == 1)
    def _():
      pltpu.make_async_copy(tab_hbm.at[sbuf.at[islot]], rows.at[1],
                            gsem1).wait()

  def start_scatter(islot, rslot):
    @pl.when(rslot == 0)
    def _():
      pltpu.async_copy(rows.at[0], acc_sh.at[dbuf.at[islot]], ssem0,
                       add=True)

    @pl.when(rslot == 1)
    def _():
      pltpu.async_copy(rows.at[1], acc_sh.at[dbuf.at[islot]], ssem1,
                       add=True)

  def wait_scatter(islot, rslot):
    @pl.when(rslot == 0)
    def _():
      pltpu.make_async_copy(rows.at[0], acc_sh.at[dbuf.at[islot]],
                            ssem0).wait()

    @pl.when(rslot == 1)
    def _():
      pltpu.make_async_copy(rows.at[1], acc_sh.at[dbuf.at[islot]],
                            ssem1).wait()

  fetch_idx(0, 0)
  fetch_idx(1, 1)
  wait_idx(0, 0)
  start_gather(0, jnp.int32(0))

  def ebody(i, carry):
    islot = lax.rem(i, 3)
    rslot = lax.rem(i, 2)

    @pl.when(i >= 1)
    def _():
      wait_scatter(lax.rem(i + 2, 3), lax.rem(i + 1, 2))

    @pl.when(i + 1 < _NCH)
    def _():
      wait_idx(i + 1, lax.rem(i + 1, 3))
      start_gather(lax.rem(i + 1, 3), lax.rem(i + 1, 2))

    @pl.when(i + 2 < _NCH)
    def _():
      fetch_idx(i + 2, lax.rem(i + 2, 3))

    wait_gather(islot, rslot)
    pending = None
    for g in range(_GRP):
      sl = pl.ds(g * _LN, _LN)
      nv = (plsc.load_gather(dinv, [sbuf[islot, sl]])
            * ebuf[islot, sl]
            * plsc.load_gather(dinv, [dbuf[islot, sl]]))
      for r in range(_LN):
        scale = jnp.take_along_axis(
            nv, jnp.full((_LN,), r, jnp.int32), axis=0,
            mode="promise_in_bounds")
        row = g * _LN + r
        vals = [rows[rslot, row, pl.ds(k * _LN, _LN)]
                for k in range(_F // _LN)]
        if pending is not None:
          prow, pprods = pending
          for k in range(_F // _LN):
            rows[rslot, prow, pl.ds(k * _LN, _LN)] = pprods[k]
        pending = (row, [v * scale for v in vals])
    prow, pprods = pending
    for k in range(_F // _LN):
      rows[rslot, prow, pl.ds(k * _LN, _LN)] = pprods[k]
    start_scatter(islot, rslot)
    return carry

  lax.fori_loop(0, _NCH, ebody, 0)
  wait_scatter(jnp.int32((_NCH - 1) % 3), jnp.int32((_NCH - 1) % 2))
  plsc.subcore_barrier()

  def obody(i, carry):
    rsl = pl.ds(s * _RPT + i * _ZB, _ZB)
    pltpu.sync_copy(acc_sh.at[rsl], out_hbm.at[c, rsl])
    return carry

  lax.fori_loop(0, _RPT // _ZB, obody, 0)


def _dinv_body(dp_ref, o_ref):
  o_ref[...] = lax.rsqrt(dp_ref[0] + dp_ref[1] + 1.0)


_dinv = pl.pallas_call(
    _dinv_body,
    out_shape=jax.ShapeDtypeStruct((_NP // 128, 128), jnp.float32),
)


_BLK = 2000
_NBLK = _N // _BLK


def _mm_body(p_ref, x_ref, dp_ref, w1_ref, b1_ref, w2_ref, g_ref):
  deg = dp_ref[0] + dp_ref[1] + 1.0
  y1 = p_ref[0] + p_ref[1] + x_ref[...] / deg
  h = jnp.maximum(
      jnp.dot(y1, w1_ref[...], preferred_element_type=jnp.float32)
      + b1_ref[...], 0.0)
  g_ref[...] = jnp.dot(h, w2_ref[...], preferred_element_type=jnp.float32)


_mm = pl.pallas_call(
    _mm_body,
    grid=(_NBLK,),
    in_specs=[
        pl.BlockSpec((2, _BLK, _F), lambda i: (0, i, 0)),
        pl.BlockSpec((_BLK, _F), lambda i: (i, 0)),
        pl.BlockSpec((2, _BLK, 1), lambda i: (0, i, 0)),
        pl.BlockSpec((128, 256), lambda i: (0, 0)),
        pl.BlockSpec((1, 256), lambda i: (0, 0)),
        pl.BlockSpec((256, 128), lambda i: (0, 0)),
    ],
    out_specs=pl.BlockSpec((_BLK, _F), lambda i: (i, 0)),
    out_shape=jax.ShapeDtypeStruct((_N, _F), jnp.float32),
)


def _fin_body(q_ref, g_ref, dp_ref, b2_ref, o_ref):
  deg = dp_ref[0] + dp_ref[1] + 1.0
  o_ref[...] = q_ref[0] + q_ref[1] + g_ref[...] / deg + b2_ref[...]


_fin = pl.pallas_call(
    _fin_body,
    grid=(_NBLK,),
    in_specs=[
        pl.BlockSpec((2, _BLK, _F), lambda i: (0, i, 0)),
        pl.BlockSpec((_BLK, _F), lambda i: (i, 0)),
        pl.BlockSpec((2, _BLK, 1), lambda i: (0, i, 0)),
        pl.BlockSpec((1, _F), lambda i: (0, 0)),
    ],
    out_specs=pl.BlockSpec((_BLK, _F), lambda i: (i, 0)),
    out_shape=jax.ShapeDtypeStruct((_N, _F), jnp.float32),
)


def kernel(x, edge_index, edge_weight, W1, b1, W2, b2):
  src = edge_index[0].astype(jnp.int32)
  dst = edge_index[1].astype(jnp.int32)
  ew = edge_weight.astype(jnp.float32)
  degp = _deg(dst, ew)
  dinv = _dinv(degp.reshape(2, _NP // 128, 128)).reshape(_NP)
  p = _agg(x, src, dst, ew, dinv)
  dpcol = degp[:, :_N, None]
  g = _mm(p, x, dpcol, W1, b1.reshape(1, -1), W2)
  q = _agg(g, src, dst, ew, dinv)
  return _fin(q, g, dpcol, b2.reshape(1, -1))


# speedup vs baseline: 1.0025x; 1.0025x over previous
import functools

import jax
import jax.numpy as jnp
from jax import lax
from jax.experimental import pallas as pl
from jax.experimental.pallas import tpu as pltpu
from jax.experimental.pallas import tpu_sc as plsc

_N = 10000
_E = 320000
_F = 128
_NP = 10240
_NC, _NS, _LN = 2, 16, 16
_NW = _NC * _NS
_EPW = _E // _NW
_CH = 80
_NCH = _EPW // _CH
_GRP = _CH // _LN
_PPT = _NP // _NS
_RPT = _NP // _NS
_ZB = 16

_MESH = plsc.VectorSubcoreMesh(core_axis_name="c", subcore_axis_name="s")
_SC_PARAMS = pltpu.CompilerParams(needs_layout_passes=False)


def _zero_vec_ref(ref, n_vec):
  for i in range(n_vec):
    ref[pl.ds(i * _LN, _LN)] = jnp.zeros((_LN,), jnp.float32)


@functools.partial(
    pl.kernel,
    out_type=jax.ShapeDtypeStruct((2, _NP), jnp.float32),
    mesh=_MESH,
    compiler_params=_SC_PARAMS,
    scratch_types=[
        pltpu.VMEM((3, _CH), jnp.int32),
        pltpu.VMEM((3, _CH), jnp.float32),
        pltpu.VMEM((_PPT,), jnp.float32),
        pltpu.VMEM_SHARED((_NP,), jnp.float32),
        pltpu.SemaphoreType.DMA,
    ],
)
def _deg(dst_hbm, ew_hbm, out_hbm, dbuf, ebuf, zb, deg_sh, isem):
  c = lax.axis_index("c")
  s = lax.axis_index("s")
  wid = s * _NC + c
  base0 = wid * _EPW
  _zero_vec_ref(zb, _PPT // _LN)
  pltpu.sync_copy(zb, deg_sh.at[pl.ds(s * _PPT, _PPT)])
  plsc.subcore_barrier()

  def fetch(i, slot):
    pltpu.async_copy(dst_hbm.at[pl.ds(base0 + i * _CH, _CH)],
                     dbuf.at[slot], isem)
    pltpu.async_copy(ew_hbm.at[pl.ds(base0 + i * _CH, _CH)],
                     ebuf.at[slot], isem)

  def wait_fetch(i, slot):
    pltpu.make_async_copy(dst_hbm.at[pl.ds(base0 + i * _CH, _CH)],
                          dbuf.at[slot], isem).wait()
    pltpu.make_async_copy(ew_hbm.at[pl.ds(base0 + i * _CH, _CH)],
                          ebuf.at[slot], isem).wait()

  fetch(0, 0)
  fetch(1, 1)

  def body(i, carry):
    slot = lax.rem(i, 3)
    wait_fetch(i, slot)

    @pl.when(i + 2 < _NCH)
    def _():
      fetch(i + 2, lax.rem(i + 2, 3))

    pltpu.sync_copy(ebuf.at[slot], deg_sh.at[dbuf.at[slot]], add=True)
    return carry

  lax.fori_loop(0, _NCH, body, 0)
  plsc.subcore_barrier()
  pltpu.sync_copy(deg_sh.at[pl.ds(s * _PPT, _PPT)],
                  out_hbm.at[c, pl.ds(s * _PPT, _PPT)])


@functools.partial(
    pl.kernel,
    out_type=jax.ShapeDtypeStruct((2, _NP, _F), jnp.float32),
    mesh=_MESH,
    compiler_params=_SC_PARAMS,
    scratch_types=[
        pltpu.VMEM((3, _CH), jnp.int32),
        pltpu.VMEM((3, _CH), jnp.int32),
        pltpu.VMEM((3, _CH), jnp.float32),
        pltpu.VMEM((2, _CH, _F), jnp.float32),
        pltpu.VMEM((_NP,), jnp.float32),
        pltpu.VMEM((_ZB, _F), jnp.float32),
        pltpu.VMEM_SHARED((_NP, _F), jnp.float32),
        pltpu.SemaphoreType.DMA,
        pltpu.SemaphoreType.DMA,
        pltpu.SemaphoreType.DMA,
        pltpu.SemaphoreType.DMA,
        pltpu.SemaphoreType.DMA,
    ],
)
def _agg(tab_hbm, src_hbm, dst_hbm, ew_hbm, dinv_hbm, out_hbm,
         sbuf, dbuf, ebuf, rows, dinv, zrows, acc_sh, isem,
         gsem0, gsem1, ssem0, ssem1):
  c = lax.axis_index("c")
  s = lax.axis_index("s")
  wid = s * _NC + c
  base0 = wid * _EPW

  for r in range(_ZB):
    for k in range(_F // _LN):
      zrows[r, pl.ds(k * _LN, _LN)] = jnp.zeros((_LN,), jnp.float32)

  def zbody(i, carry):
    pltpu.sync_copy(zrows, acc_sh.at[pl.ds(s * _RPT + i * _ZB, _ZB)])
    return carry

  lax.fori_loop(0, _RPT // _ZB, zbody, 0)

  pltpu.sync_copy(dinv_hbm, dinv)
  plsc.subcore_barrier()

  def fetch_idx(i, slot):
    pltpu.async_copy(src_hbm.at[pl.ds(base0 + i * _CH, _CH)],
                     sbuf.at[slot], isem)
    pltpu.async_copy(dst_hbm.at[pl.ds(base0 + i * _CH, _CH)],
                     dbuf.at[slot], isem)
    pltpu.async_copy(ew_hbm.at[pl.ds(base0 + i * _CH, _CH)],
                     ebuf.at[slot], isem)

  def wait_idx(i, slot):
    pltpu.make_async_copy(src_hbm.at[pl.ds(base0 + i * _CH, _CH)],
                          sbuf.at[slot], isem).wait()
    pltpu.make_async_copy(dst_hbm.at[pl.ds(base0 + i * _CH, _CH)],
                          dbuf.at[slot], isem).wait()
    pltpu.make_async_copy(ew_hbm.at[pl.ds(base0 + i * _CH, _CH)],
                          ebuf.at[slot], isem).wait()

  def start_gather(islot, rslot):
    @pl.when(rslot == 0)
    def _():
      pltpu.async_copy(tab_hbm.at[sbuf.at[islot]], rows.at[0], gsem0)

    @pl.when(rslot == 1)
    def _():
      pltpu.async_copy(tab_hbm.at[sbuf.at[islot]], rows.at[1], gsem1)

  def wait_gather(islot, rslot):
    @pl.when(rslot == 0)
    def _():
      pltpu.make_async_copy(tab_hbm.at[sbuf.at[islot]], rows.at[0],
                            gsem0).wait()

    @pl.when(rslot == 1)
    def _():
      pltpu.make_async_copy(tab_hbm.at[sbuf.at[islot]], rows.at[1],
                            gsem1).wait()

  def start_scatter(islot, rslot):
    @pl.when(rslot == 0)
    def _():
      pltpu.async_copy(rows.at[0], acc_sh.at[dbuf.at[islot]], ssem0,
                       add=True)

    @pl.when(rslot == 1)
    def _():
      pltpu.async_copy(rows.at[1], acc_sh.at[dbuf.at[islot]], ssem1,
                       add=True)

  def wait_scatter(islot, rslot):
    @pl.when(rslot == 0)
    def _():
      pltpu.make_async_copy(rows.at[0], acc_sh.at[dbuf.at[islot]],
                            ssem0).wait()

    @pl.when(rslot == 1)
    def _():
      pltpu.make_async_copy(rows.at[1], acc_sh.at[dbuf.at[islot]],
                            ssem1).wait()

  fetch_idx(0, 0)
  fetch_idx(1, 1)
  wait_idx(0, 0)
  start_gather(0, jnp.int32(0))

  def ebody(i, carry):
    islot = lax.rem(i, 3)
    rslot = lax.rem(i, 2)

    @pl.when(i >= 1)
    def _():
      wait_scatter(lax.rem(i + 2, 3), lax.rem(i + 1, 2))

    @pl.when(i + 1 < _NCH)
    def _():
      wait_idx(i + 1, lax.rem(i + 1, 3))
      start_gather(lax.rem(i + 1, 3), lax.rem(i + 1, 2))

    @pl.when(i + 2 < _NCH)
    def _():
      fetch_idx(i + 2, lax.rem(i + 2, 3))

    wait_gather(islot, rslot)
    pending = None
    for g in range(_GRP):
      sl = pl.ds(g * _LN, _LN)
      nv = (plsc.load_gather(dinv, [sbuf[islot, sl]])
            * ebuf[islot, sl]
            * plsc.load_gather(dinv, [dbuf[islot, sl]]))
      for r in range(_LN):
        scale = jnp.take_along_axis(
            nv, jnp.full((_LN,), r, jnp.int32), axis=0,
            mode="promise_in_bounds")
        row = g * _LN + r
        vals = [rows[rslot, row, pl.ds(k * _LN, _LN)]
                for k in range(_F // _LN)]
        if pending is not None:
          prow, pprods = pending
          for k in range(_F // _LN):
            rows[rslot, prow, pl.ds(k * _LN, _LN)] = pprods[k]
        pending = (row, [v * scale for v in vals])
    prow, pprods = pending
    for k in range(_F // _LN):
      rows[rslot, prow, pl.ds(k * _LN, _LN)] = pprods[k]
    start_scatter(islot, rslot)
    return carry

  lax.fori_loop(0, _NCH, ebody, 0)
  wait_scatter(jnp.int32((_NCH - 1) % 3), jnp.int32((_NCH - 1) % 2))
  plsc.subcore_barrier()

  def obody(i, carry):
    rsl = pl.ds(s * _RPT + i * _ZB, _ZB)
    pltpu.sync_copy(acc_sh.at[rsl], out_hbm.at[c, rsl])
    return carry

  lax.fori_loop(0, _RPT // _ZB, obody, 0)


def _dinv_body(dp_ref, o_ref):
  o_ref[...] = lax.rsqrt(dp_ref[0] + dp_ref[1] + 1.0)


_dinv = pl.pallas_call(
    _dinv_body,
    out_shape=jax.ShapeDtypeStruct((_NP // 128, 128), jnp.float32),
)


_BLK = 2000
_NBLK = _N // _BLK


def _mm_body(p_ref, x_ref, dp_ref, w1_ref, b1_ref, w2_ref, g_ref):
  deg = dp_ref[0] + dp_ref[1] + 1.0
  y1 = p_ref[0] + p_ref[1] + x_ref[...] / deg
  h = jnp.maximum(
      jnp.dot(y1, w1_ref[...], preferred_element_type=jnp.float32)
      + b1_ref[...], 0.0)
  g_ref[...] = jnp.dot(h, w2_ref[...], preferred_element_type=jnp.float32)


_mm = pl.pallas_call(
    _mm_body,
    grid=(_NBLK,),
    in_specs=[
        pl.BlockSpec((2, _BLK, _F), lambda i: (0, i, 0)),
        pl.BlockSpec((_BLK, _F), lambda i: (i, 0)),
        pl.BlockSpec((2, _BLK, 1), lambda i: (0, i, 0)),
        pl.BlockSpec((128, 256), lambda i: (0, 0)),
        pl.BlockSpec((1, 256), lambda i: (0, 0)),
        pl.BlockSpec((256, 128), lambda i: (0, 0)),
    ],
    out_specs=pl.BlockSpec((_BLK, _F), lambda i: (i, 0)),
    out_shape=jax.ShapeDtypeStruct((_N, _F), jnp.float32),
)


def _fin_body(q_ref, g_ref, dp_ref, b2_ref, o_ref):
  deg = dp_ref[0] + dp_ref[1] + 1.0
  o_ref[...] = q_ref[0] + q_ref[1] + g_ref[...] / deg + b2_ref[...]


_fin = pl.pallas_call(
    _fin_body,
    grid=(_NBLK,),
    in_specs=[
        pl.BlockSpec((2, _BLK, _F), lambda i: (0, i, 0)),
        pl.BlockSpec((_BLK, _F), lambda i: (i, 0)),
        pl.BlockSpec((2, _BLK, 1), lambda i: (0, i, 0)),
        pl.BlockSpec((1, _F), lambda i: (0, 0)),
    ],
    out_specs=pl.BlockSpec((_BLK, _F), lambda i: (i, 0)),
    out_shape=jax.ShapeDtypeStruct((_N, _F), jnp.float32),
)


def kernel(x, edge_index, edge_weight, W1, b1, W2, b2):
  src = edge_index[0].astype(jnp.int32)
  dst = edge_index[1].astype(jnp.int32)
  ew = edge_weight.astype(jnp.float32)
  degp = _deg(dst, ew)
  dinv = _dinv(degp.reshape(2, _NP // 128, 128)).reshape(_NP)
  p = _agg(x, src, dst, ew, dinv)
  dpcol = degp[:, :_N, None]
  g = _mm(p, x, dpcol, W1, b1.reshape(1, -1), W2)
  q = _agg(g, src, dst, ew, dinv)
  return _fin(q, g, dpcol, b2.reshape(1, -1))
